# Initial kernel scaffold; baseline (speedup 1.0000x reference)
import jax
import jax.numpy as jnp
from jax.experimental import pallas as pl


def kernel(h, r, tp, tn, table, W, edge_index):
    raise NotImplementedError("write your pallas kernel here")


# speedup vs baseline: 2.0827x; 2.0827x over previous
import functools

import jax
import jax.numpy as jnp
from jax import lax
from jax.experimental import pallas as pl
from jax.experimental.pallas import tpu as pltpu
from jax.experimental.pallas import tpu_sc as plsc

N_ENT = 100000
DIM = 64
N_EDGES = 1600000
BATCH = 16384

NC, NS = 2, 16
NW = NC * NS
NB = 25600
NPASS = 2
NPAD = NC * NPASS * NB
CH = 128
EPT = 100096
EPAD = NS * EPT
ROWS_PER_TILE = NB // NS
ZR = 160

_mesh = plsc.VectorSubcoreMesh(core_axis_name="c", subcore_axis_name="s")


@functools.partial(
    pl.kernel,
    mesh=_mesh,
    out_type=[
        jax.ShapeDtypeStruct((NPAD, DIM), jnp.float32),
        jax.ShapeDtypeStruct((NPAD,), jnp.float32),
    ],
    scratch_types=[
        pltpu.VMEM((CH,), jnp.int32),
        pltpu.VMEM((CH,), jnp.int32),
        pltpu.VMEM((CH,), jnp.int32),
        pltpu.VMEM((CH, DIM), jnp.float32),
        pltpu.VMEM((CH,), jnp.float32),
        pltpu.VMEM((ZR, DIM), jnp.float32),
        pltpu.VMEM((ZR,), jnp.float32),
        pltpu.VMEM_SHARED((NB + 8, DIM), jnp.float32),
        pltpu.VMEM_SHARED((NB + 8,), jnp.float32),
        pltpu.SemaphoreType.DMA,
    ],
    compiler_params=pltpu.CompilerParams(use_tc_tiling_on_sc=False),
)
def _sc_scatter(table_hbm, src_hbm, dst_hbm, agg_hbm, deg_hbm,
                src_v, dst_v, lidx_v, rows_v, ones_v, zrow_v, zdeg_v,
                agg_s, deg_s, sem):
    c = lax.axis_index("c")
    s = lax.axis_index("s")
    z16 = jnp.zeros((16,), jnp.float32)
    o16 = jnp.ones((16,), jnp.float32)

    for i in range(CH // 16):
        ones_v[pl.ds(i * 16, 16)] = o16
    for i in range(ZR // 16):
        zdeg_v[pl.ds(i * 16, 16)] = z16

    def zrow_body(i, carry):
        for k in range(DIM // 16):
            zrow_v[i, pl.ds(k * 16, 16)] = z16
        return carry

    lax.fori_loop(0, ZR, zrow_body, 0)

    for p in range(NPASS):
        base = (c * NPASS + p) * NB
        tile_row0 = s * ROWS_PER_TILE

        def clear_body(j, carry):
            pltpu.sync_copy(zrow_v, agg_s.at[pl.ds(tile_row0 + j * ZR, ZR)])
            pltpu.sync_copy(zdeg_v, deg_s.at[pl.ds(tile_row0 + j * ZR, ZR)])
            return carry

        lax.fori_loop(0, ROWS_PER_TILE // ZR, clear_body, 0)
        plsc.subcore_barrier()

        def edge_body(t, carry):
            off = s * EPT + t * CH
            pltpu.sync_copy(src_hbm.at[pl.ds(off, CH)], src_v)
            pltpu.sync_copy(dst_hbm.at[pl.ds(off, CH)], dst_v)

            def lidx_body(i, carry2):
                v = dst_v[pl.ds(i * 16, 16)]
                inr = (v >= base) & (v < base + NB)
                lidx_v[pl.ds(i * 16, 16)] = jnp.where(inr, v - base,
                                                      jnp.int32(NB))
                return carry2

            lax.fori_loop(0, CH // 16, lidx_body, 0)
            pltpu.async_copy(table_hbm.at[src_v], rows_v, sem).wait()
            pltpu.sync_copy(rows_v, agg_s.at[lidx_v], add=True)
            pltpu.sync_copy(ones_v, deg_s.at[lidx_v], add=True)
            return carry

        lax.fori_loop(0, EPT // CH, edge_body, 0)
        plsc.subcore_barrier()

        pltpu.sync_copy(agg_s.at[pl.ds(tile_row0, ROWS_PER_TILE)],
                        agg_hbm.at[pl.ds(base + tile_row0, ROWS_PER_TILE)])
        pltpu.sync_copy(deg_s.at[pl.ds(tile_row0, ROWS_PER_TILE)],
                        deg_hbm.at[pl.ds(base + tile_row0, ROWS_PER_TILE)])
        plsc.subcore_barrier()


_QB = 3 * BATCH
_QPT = _QB // NW


@functools.partial(
    pl.kernel,
    mesh=_mesh,
    out_type=jax.ShapeDtypeStruct((_QB, DIM), jnp.float32),
    scratch_types=[
        pltpu.VMEM((CH,), jnp.int32),
        pltpu.VMEM((CH, DIM), jnp.float32),
        pltpu.SemaphoreType.DMA,
    ],
    compiler_params=pltpu.CompilerParams(use_tc_tiling_on_sc=False),
)
def _sc_gather(emb_hbm, q_hbm, out_hbm, idx_v, rows_v, sem):
    c = lax.axis_index("c")
    s = lax.axis_index("s")
    wid = s * NC + c
    base = wid * _QPT

    def body(t, carry):
        off = base + t * CH
        pltpu.sync_copy(q_hbm.at[pl.ds(off, CH)], idx_v)
        pltpu.async_copy(emb_hbm.at[idx_v], rows_v, sem).wait()
        pltpu.sync_copy(rows_v, out_hbm.at[pl.ds(off, CH)])
        return carry

    lax.fori_loop(0, _QPT // CH, body, 0)


def _gcn_body(tab_ref, agg_ref, deg_ref, w_ref, out_ref):
    deg = jnp.maximum(deg_ref[...], 1.0)
    x = tab_ref[...] + agg_ref[...] / deg
    out_ref[...] = jnp.tanh(
        jnp.dot(x, w_ref[...], preferred_element_type=jnp.float32))


_GBLK = 1024


def _gcn_dense(tab, agg, deg, W):
    grid = (NPAD // _GBLK,)
    return pl.pallas_call(
        _gcn_body,
        grid=grid,
        in_specs=[
            pl.BlockSpec((_GBLK, DIM), lambda i: (i, 0)),
            pl.BlockSpec((_GBLK, DIM), lambda i: (i, 0)),
            pl.BlockSpec((_GBLK, 1), lambda i: (i, 0)),
            pl.BlockSpec((DIM, DIM), lambda i: (0, 0)),
        ],
        out_specs=pl.BlockSpec((_GBLK, DIM), lambda i: (i, 0)),
        out_shape=jax.ShapeDtypeStruct((NPAD, DIM), jnp.float32),
    )(tab, agg, deg, W)


def _score_body(he_ref, tpe_ref, tne_ref, out_ref):
    h = he_ref[...]
    up = jnp.sum((h - tpe_ref[...]) ** 2, axis=1, keepdims=True)
    un = jnp.sum((h - tne_ref[...]) ** 2, axis=1, keepdims=True)
    d = un - up
    out_ref[...] = jnp.maximum(d, 0.0) + jnp.log1p(jnp.exp(-jnp.abs(d)))


_SBLK = 2048


def _score(h_e, tp_e, tn_e):
    grid = (BATCH // _SBLK,)
    return pl.pallas_call(
        _score_body,
        grid=grid,
        in_specs=[
            pl.BlockSpec((_SBLK, DIM), lambda i: (i, 0)),
            pl.BlockSpec((_SBLK, DIM), lambda i: (i, 0)),
            pl.BlockSpec((_SBLK, DIM), lambda i: (i, 0)),
        ],
        out_specs=pl.BlockSpec((_SBLK, 1), lambda i: (i, 0)),
        out_shape=jax.ShapeDtypeStruct((BATCH, 1), jnp.float32),
    )(h_e, tp_e, tn_e)


@jax.jit
def kernel(h, r, tp, tn, table, W, edge_index):
    src = edge_index[0].astype(jnp.int32)
    dst = edge_index[1].astype(jnp.int32)
    pad = EPAD - N_EDGES
    src_p = jnp.concatenate([src, jnp.zeros((pad,), jnp.int32)])
    dst_p = jnp.concatenate([dst, jnp.full((pad,), NPAD, jnp.int32)])

    agg, deg = _sc_scatter(table, src_p, dst_p)

    tab_p = jnp.concatenate(
        [table, jnp.zeros((NPAD - N_ENT, DIM), table.dtype)], axis=0)
    emb = _gcn_dense(tab_p, agg, deg.reshape(NPAD, 1), W)

    q = jnp.concatenate([h, tp, tn]).astype(jnp.int32)
    rows = _sc_gather(emb, q)
    h_e = rows[:BATCH]
    tp_e = rows[BATCH:2 * BATCH]
    tn_e = rows[2 * BATCH:]
    return _score(h_e, tp_e, tn_e)[:, 0]


# speedup vs baseline: 2.1750x; 1.0443x over previous
import functools

import jax
import jax.numpy as jnp
from jax import lax
from jax.experimental import pallas as pl
from jax.experimental.pallas import tpu as pltpu
from jax.experimental.pallas import tpu_sc as plsc

N_ENT = 100000
DIM = 64
N_EDGES = 1600000
BATCH = 16384

NC, NS = 2, 16
NW = NC * NS
NB = 25088
NPASS = 2
NPAD = NC * NPASS * NB
CH = 128
EPT = 100096
EPAD = NS * EPT
ROWS_PER_TILE = NB // NS
ZR = 112

_mesh = plsc.VectorSubcoreMesh(core_axis_name="c", subcore_axis_name="s")


@functools.partial(
    pl.kernel,
    mesh=_mesh,
    out_type=[
        jax.ShapeDtypeStruct((NPAD, DIM), jnp.float32),
        jax.ShapeDtypeStruct((NPAD,), jnp.float32),
    ],
    scratch_types=[
        pltpu.VMEM((CH,), jnp.int32),
        pltpu.VMEM((CH,), jnp.int32),
        pltpu.VMEM((CH,), jnp.int32),
        pltpu.VMEM((CH, DIM), jnp.float32),
        pltpu.VMEM((CH,), jnp.int32),
        pltpu.VMEM((CH,), jnp.int32),
        pltpu.VMEM((CH,), jnp.int32),
        pltpu.VMEM((CH, DIM), jnp.float32),
        pltpu.VMEM((CH,), jnp.float32),
        pltpu.VMEM((ZR, DIM), jnp.float32),
        pltpu.VMEM((ZR,), jnp.float32),
        pltpu.VMEM_SHARED((NB + 8, DIM), jnp.float32),
        pltpu.VMEM_SHARED((NB + 8,), jnp.float32),
        pltpu.SemaphoreType.DMA,
        pltpu.SemaphoreType.DMA,
    ],
    compiler_params=pltpu.CompilerParams(use_tc_tiling_on_sc=False),
)
def _sc_scatter(table_hbm, src_hbm, dst_hbm, agg_hbm, deg_hbm,
                src_a, dst_a, lidx_a, rows_a, src_b, dst_b, lidx_b, rows_b,
                ones_v, zrow_v, zdeg_v, agg_s, deg_s, sem_a, sem_b):
    c = lax.axis_index("c")
    s = lax.axis_index("s")
    z16 = jnp.zeros((16,), jnp.float32)
    o16 = jnp.ones((16,), jnp.float32)

    for i in range(CH // 16):
        ones_v[pl.ds(i * 16, 16)] = o16
    for i in range(ZR // 16):
        zdeg_v[pl.ds(i * 16, 16)] = z16

    def zrow_body(i, carry):
        for k in range(DIM // 16):
            zrow_v[i, pl.ds(k * 16, 16)] = z16
        return carry

    lax.fori_loop(0, ZR, zrow_body, 0)

    for p in range(NPASS):
        base = (c * NPASS + p) * NB
        tile_row0 = s * ROWS_PER_TILE

        def clear_body(j, carry):
            pltpu.sync_copy(zrow_v, agg_s.at[pl.ds(tile_row0 + j * ZR, ZR)])
            pltpu.sync_copy(zdeg_v, deg_s.at[pl.ds(tile_row0 + j * ZR, ZR)])
            return carry

        lax.fori_loop(0, ROWS_PER_TILE // ZR, clear_body, 0)
        plsc.subcore_barrier()

        nch = EPT // CH

        def load_idx(t, src_r, dst_r, lidx_r):
            off = s * EPT + t * CH
            pltpu.sync_copy(src_hbm.at[pl.ds(off, CH)], src_r)
            pltpu.sync_copy(dst_hbm.at[pl.ds(off, CH)], dst_r)

            def lidx_body(i, carry2):
                v = dst_r[pl.ds(i * 16, 16)]
                inr = (v >= base) & (v < base + NB)
                lidx_r[pl.ds(i * 16, 16)] = jnp.where(inr, v - base,
                                                      jnp.int32(NB))
                return carry2

            lax.fori_loop(0, CH // 16, lidx_body, 0)

        def wait_gather(rows_r, sem_r):
            pltpu.make_async_copy(table_hbm.at[pl.ds(0, CH)], rows_r,
                                  sem_r).wait()

        load_idx(0, src_a, dst_a, lidx_a)
        pltpu.async_copy(table_hbm.at[src_a], rows_a, sem_a)

        def edge_body(u, carry):
            c0 = 2 * u
            load_idx(c0 + 1, src_b, dst_b, lidx_b)
            pltpu.async_copy(table_hbm.at[src_b], rows_b, sem_b)
            wait_gather(rows_a, sem_a)
            pltpu.sync_copy(rows_a, agg_s.at[lidx_a], add=True)
            pltpu.sync_copy(ones_v, deg_s.at[lidx_a], add=True)
            load_idx(jnp.minimum(c0 + 2, nch - 1), src_a, dst_a, lidx_a)
            pltpu.async_copy(table_hbm.at[src_a], rows_a, sem_a)
            wait_gather(rows_b, sem_b)
            pltpu.sync_copy(rows_b, agg_s.at[lidx_b], add=True)
            pltpu.sync_copy(ones_v, deg_s.at[lidx_b], add=True)
            return carry

        lax.fori_loop(0, nch // 2, edge_body, 0)
        wait_gather(rows_a, sem_a)
        plsc.subcore_barrier()

        pltpu.sync_copy(agg_s.at[pl.ds(tile_row0, ROWS_PER_TILE)],
                        agg_hbm.at[pl.ds(base + tile_row0, ROWS_PER_TILE)])
        pltpu.sync_copy(deg_s.at[pl.ds(tile_row0, ROWS_PER_TILE)],
                        deg_hbm.at[pl.ds(base + tile_row0, ROWS_PER_TILE)])
        plsc.subcore_barrier()


_QB = 3 * BATCH
_QPT = _QB // NW


@functools.partial(
    pl.kernel,
    mesh=_mesh,
    out_type=jax.ShapeDtypeStruct((_QB, DIM), jnp.float32),
    scratch_types=[
        pltpu.VMEM((CH,), jnp.int32),
        pltpu.VMEM((CH, DIM), jnp.float32),
        pltpu.SemaphoreType.DMA,
    ],
    compiler_params=pltpu.CompilerParams(use_tc_tiling_on_sc=False),
)
def _sc_gather(emb_hbm, q_hbm, out_hbm, idx_v, rows_v, sem):
    c = lax.axis_index("c")
    s = lax.axis_index("s")
    wid = s * NC + c
    base = wid * _QPT

    def body(t, carry):
        off = base + t * CH
        pltpu.sync_copy(q_hbm.at[pl.ds(off, CH)], idx_v)
        pltpu.async_copy(emb_hbm.at[idx_v], rows_v, sem).wait()
        pltpu.sync_copy(rows_v, out_hbm.at[pl.ds(off, CH)])
        return carry

    lax.fori_loop(0, _QPT // CH, body, 0)


def _gcn_body(tab_ref, agg_ref, deg_ref, w_ref, out_ref):
    deg = jnp.maximum(deg_ref[...], 1.0)
    x = tab_ref[...] + agg_ref[...] / deg
    out_ref[...] = jnp.tanh(
        jnp.dot(x, w_ref[...], preferred_element_type=jnp.float32))


_GBLK = 1024


def _gcn_dense(tab, agg, deg, W):
    grid = (NPAD // _GBLK,)
    return pl.pallas_call(
        _gcn_body,
        grid=grid,
        in_specs=[
            pl.BlockSpec((_GBLK, DIM), lambda i: (i, 0)),
            pl.BlockSpec((_GBLK, DIM), lambda i: (i, 0)),
            pl.BlockSpec((_GBLK, 1), lambda i: (i, 0)),
            pl.BlockSpec((DIM, DIM), lambda i: (0, 0)),
        ],
        out_specs=pl.BlockSpec((_GBLK, DIM), lambda i: (i, 0)),
        out_shape=jax.ShapeDtypeStruct((NPAD, DIM), jnp.float32),
    )(tab, agg, deg, W)


def _score_body(he_ref, tpe_ref, tne_ref, out_ref):
    h = he_ref[...]
    up = jnp.sum((h - tpe_ref[...]) ** 2, axis=1, keepdims=True)
    un = jnp.sum((h - tne_ref[...]) ** 2, axis=1, keepdims=True)
    d = un - up
    out_ref[...] = jnp.maximum(d, 0.0) + jnp.log1p(jnp.exp(-jnp.abs(d)))


_SBLK = 2048


def _score(h_e, tp_e, tn_e):
    grid = (BATCH // _SBLK,)
    return pl.pallas_call(
        _score_body,
        grid=grid,
        in_specs=[
            pl.BlockSpec((_SBLK, DIM), lambda i: (i, 0)),
            pl.BlockSpec((_SBLK, DIM), lambda i: (i, 0)),
            pl.BlockSpec((_SBLK, DIM), lambda i: (i, 0)),
        ],
        out_specs=pl.BlockSpec((_SBLK, 1), lambda i: (i, 0)),
        out_shape=jax.ShapeDtypeStruct((BATCH, 1), jnp.float32),
    )(h_e, tp_e, tn_e)


@jax.jit
def kernel(h, r, tp, tn, table, W, edge_index):
    src = edge_index[0].astype(jnp.int32)
    dst = edge_index[1].astype(jnp.int32)
    pad = EPAD - N_EDGES
    src_p = jnp.concatenate([src, jnp.zeros((pad,), jnp.int32)])
    dst_p = jnp.concatenate([dst, jnp.full((pad,), NPAD, jnp.int32)])

    agg, deg = _sc_scatter(table, src_p, dst_p)

    tab_p = jnp.concatenate(
        [table, jnp.zeros((NPAD - N_ENT, DIM), table.dtype)], axis=0)
    emb = _gcn_dense(tab_p, agg, deg.reshape(NPAD, 1), W)

    q = jnp.concatenate([h, tp, tn]).astype(jnp.int32)
    rows = _sc_gather(emb, q)
    h_e = rows[:BATCH]
    tp_e = rows[BATCH:2 * BATCH]
    tn_e = rows[2 * BATCH:]
    return _score(h_e, tp_e, tn_e)[:, 0]
